# baseline (speedup 1.0000x reference)
import functools

import jax
import jax.numpy as jnp
from jax import lax
from jax.experimental import pallas as pl
from jax.experimental.pallas import tpu as pltpu
from jax.experimental.pallas import tpu_sc as plsc

NB = 8192
NBINS = 2 * NB
NSUB = 4
CLAMP = 12.0
SCALE = NB / (2.0 * CLAMP)
NW = 32
LANES = 16
UNROLL = 8

ROWS = 16384
COLS = 512
CH_ROWS = 16
CH_ELEMS = CH_ROWS * COLS
VPC = CH_ELEMS // LANES
VPR = COLS // LANES

HROWS = 64
HCOLS = 128


def _sc_hist_body(x_hbm, t_hbm, out_hbm, xb0, xb1, tb0, tb1, hsub, hout,
                  sx0, sx1, st0, st1):
    wid = lax.axis_index("s") * 2 + lax.axis_index("c")
    rows_per_w = ROWS // NW
    row_base = wid * rows_per_w
    nchunks = rows_per_w // CH_ROWS
    xbufs = (xb0, xb1)
    tbufs = (tb0, tb1)
    sxs = (sx0, sx1)
    sts = (st0, st1)

    zeros = jnp.zeros((LANES,), jnp.float32)

    @plsc.parallel_loop(0, NBINS * NSUB // LANES, 1, unroll=4)
    def _zero(i):
        hsub[pl.ds(i * LANES, LANES)] = zeros

    ones = jnp.ones((LANES,), jnp.float32)
    lane = lax.broadcasted_iota(jnp.int32, (LANES,), 0)
    lslots = (lane & 3, (lane + 2) & 3)

    def x_mul_add(xv):
        y = xv * SCALE + (CLAMP * SCALE)
        y = jnp.maximum(y, 0.0)
        return jnp.minimum(y, float(NB - 1))

    def start_chunk(cidx, b):
        row0 = row_base + cidx * CH_ROWS
        pltpu.make_async_copy(
            x_hbm.at[pl.ds(row0, CH_ROWS)], xbufs[b], sxs[b]).start()
        pltpu.make_async_copy(
            t_hbm.at[pl.ds(row0, CH_ROWS)], tbufs[b], sts[b]).start()

    def wait_chunk(b):
        pltpu.make_async_copy(
            x_hbm.at[pl.ds(row_base, CH_ROWS)], xbufs[b], sxs[b]).wait()
        pltpu.make_async_copy(
            t_hbm.at[pl.ds(row_base, CH_ROWS)], tbufs[b], sts[b]).wait()

    def process_chunk(b):
        xbuf = xbufs[b]
        tbuf = tbufs[b]

        @plsc.parallel_loop(0, VPC // UNROLL, 1, unroll=2)
        def _vecs(i):
            for u in range(UNROLL):
                v = i * UNROLL + u
                r = v // VPR
                c = (v % VPR) * LANES
                xv = xbuf[r, pl.ds(c, LANES)]
                tv = tbuf[r, pl.ds(c, LANES)]
                y = x_mul_add(xv)
                bin_ = y.astype(jnp.int32) + (tv << 13)
                idx = (bin_ << 2) | lslots[u % 2]
                plsc.addupdate_scatter(hsub, [idx], ones)

    start_chunk(0, 0)

    def chunk_pair(g, carry):
        for b in range(2):
            ci = g * 2 + b

            @pl.when(ci + 1 < nchunks)
            def _():
                start_chunk(ci + 1, 1 - b)

            wait_chunk(b)
            process_chunk(b)
        return carry

    lax.fori_loop(0, nchunks // 2, chunk_pair, 0)

    @plsc.parallel_loop(0, NBINS // LANES, 1, unroll=2)
    def _merge(i):
        base4 = (i * LANES + lane) << 2
        acc = plsc.load_gather(hsub, [base4])
        acc = acc + plsc.load_gather(hsub, [base4 | 1])
        acc = acc + plsc.load_gather(hsub, [base4 | 2])
        acc = acc + plsc.load_gather(hsub, [base4 | 3])
        hout[pl.ds(i * LANES, LANES)] = acc

    pltpu.sync_copy(hout, out_hbm.at[wid])


def _combine_body(hist_ref, out_ref):
    h = hist_ref[...]
    c = jnp.sum(h, axis=0)
    cn = c[:HROWS, :]
    cp = c[HROWS:, :]
    m_tot = jnp.sum(cn)
    g_tot = jnp.sum(cp)
    n_tot = m_tot + g_tot

    row = lax.broadcasted_iota(jnp.int32, (HROWS, HCOLS), 0)
    col = lax.broadcasted_iota(jnp.int32, (HROWS, HCOLS), 1)
    b = (row * HCOLS + col).astype(jnp.float32)
    centers = -CLAMP + (b + 0.5) * (2.0 * CLAMP / NB)
    pcen = jax.nn.sigmoid(centers)

    s_pos = jnp.sum(cp * pcen)

    i1 = lax.broadcasted_iota(jnp.int32, (HCOLS, HCOLS), 0)
    j1 = lax.broadcasted_iota(jnp.int32, (HCOLS, HCOLS), 1)
    upper = (i1 <= j1).astype(jnp.float32)
    p_row = lax.dot_general(
        cn, upper, (((1,), (0,)), ((), ())),
        precision=lax.Precision.HIGHEST,
        preferred_element_type=jnp.float32,
    )
    rowsum = p_row[:, HCOLS - 1:HCOLS]
    i2 = lax.broadcasted_iota(jnp.int32, (HROWS, HROWS), 0)
    j2 = lax.broadcasted_iota(jnp.int32, (HROWS, HROWS), 1)
    strict_lower = (j2 < i2).astype(jnp.float32)
    offs = lax.dot_general(
        strict_lower, rowsum, (((1,), (0,)), ((), ())),
        precision=lax.Precision.HIGHEST,
        preferred_element_type=jnp.float32,
    )
    prefix = p_row + offs
    rank_above = m_tot - prefix

    denom = (g_tot + rank_above) * (g_tot + rank_above + cn)
    contrib = jnp.where(cn > 0.0, pcen * g_tot * cn / denom, 0.0)
    loss = 1.0 - s_pos / n_tot + jnp.sum(contrib)

    max_center = jnp.max(jnp.where(cn > 0.0, centers, -3.0 * CLAMP))
    loss = jnp.where(g_tot == 0.0, 1.0 + jax.nn.sigmoid(max_center), loss)
    out_ref[...] = jnp.reshape(loss, (1, 1))


@functools.cache
def _get_sc_hist():
    return pl.kernel(
        _sc_hist_body,
        out_type=jax.ShapeDtypeStruct((NW, NBINS), jnp.float32),
        mesh=plsc.VectorSubcoreMesh(core_axis_name="c", subcore_axis_name="s"),
        compiler_params=pltpu.CompilerParams(needs_layout_passes=False),
        scratch_types=[
            pltpu.VMEM((CH_ROWS, COLS), jnp.float32),
            pltpu.VMEM((CH_ROWS, COLS), jnp.float32),
            pltpu.VMEM((CH_ROWS, COLS), jnp.int32),
            pltpu.VMEM((CH_ROWS, COLS), jnp.int32),
            pltpu.VMEM((NBINS * NSUB,), jnp.float32),
            pltpu.VMEM((NBINS,), jnp.float32),
            pltpu.SemaphoreType.DMA,
            pltpu.SemaphoreType.DMA,
            pltpu.SemaphoreType.DMA,
            pltpu.SemaphoreType.DMA,
        ],
    )


_combine = pl.pallas_call(
    _combine_body,
    out_shape=jax.ShapeDtypeStruct((1, 1), jnp.float32),
)


def kernel(inputs, targets):
    x = inputs.reshape(ROWS, COLS)
    t = targets.reshape(ROWS, COLS)
    hist = _get_sc_hist()(x, t)
    loss = _combine(hist.reshape(NW, 2 * HROWS, HCOLS))
    return loss.reshape(())


# speedup vs baseline: 1.1512x; 1.1512x over previous
import functools

import jax
import jax.numpy as jnp
from jax import lax
from jax.experimental import pallas as pl
from jax.experimental.pallas import tpu as pltpu
from jax.experimental.pallas import tpu_sc as plsc

NB = 16384
NBINS = 2 * NB
CLAMP = 12.0
SCALE = NB / (2.0 * CLAMP)
NW = 32
LANES = 16
UNROLL = 8

ROWS = 16384
COLS = 512
CH_ROWS = 16
CH_ELEMS = CH_ROWS * COLS
VPC = CH_ELEMS // LANES
VPR = COLS // LANES

HROWS = 128
HCOLS = 128


def _sc_hist_body(x_hbm, t_hbm, out_hbm, xb0, xb1, tb0, tb1, h0, h1,
                  sx0, sx1, st0, st1):
    wid = lax.axis_index("s") * 2 + lax.axis_index("c")
    rows_per_w = ROWS // NW
    row_base = wid * rows_per_w
    nchunks = rows_per_w // CH_ROWS
    xbufs = (xb0, xb1)
    tbufs = (tb0, tb1)
    sxs = (sx0, sx1)
    sts = (st0, st1)

    zeros = jnp.zeros((LANES,), jnp.float32)

    @plsc.parallel_loop(0, NBINS // LANES, 1, unroll=4)
    def _zero(i):
        h0[pl.ds(i * LANES, LANES)] = zeros
        h1[pl.ds(i * LANES, LANES)] = zeros

    ones = jnp.ones((LANES,), jnp.float32)
    hists = (h0, h1)

    def x_mul_add(xv):
        y = xv * SCALE + (CLAMP * SCALE)
        y = jnp.maximum(y, 0.0)
        return jnp.minimum(y, float(NB - 1))

    def start_chunk(cidx, b):
        row0 = row_base + cidx * CH_ROWS
        pltpu.make_async_copy(
            x_hbm.at[pl.ds(row0, CH_ROWS)], xbufs[b], sxs[b]).start()
        pltpu.make_async_copy(
            t_hbm.at[pl.ds(row0, CH_ROWS)], tbufs[b], sts[b]).start()

    def wait_chunk(b):
        pltpu.make_async_copy(
            x_hbm.at[pl.ds(row_base, CH_ROWS)], xbufs[b], sxs[b]).wait()
        pltpu.make_async_copy(
            t_hbm.at[pl.ds(row_base, CH_ROWS)], tbufs[b], sts[b]).wait()

    def process_chunk(b):
        xbuf = xbufs[b]
        tbuf = tbufs[b]

        @plsc.parallel_loop(0, VPC // UNROLL, 1, unroll=2)
        def _vecs(i):
            for u in range(UNROLL):
                v = i * UNROLL + u
                r = v // VPR
                c = (v % VPR) * LANES
                xv = xbuf[r, pl.ds(c, LANES)]
                tv = tbuf[r, pl.ds(c, LANES)]
                y = x_mul_add(xv)
                idx = y.astype(jnp.int32) + (tv << 14)
                plsc.addupdate_scatter(hists[u % 2], [idx], ones)

    start_chunk(0, 0)

    def chunk_pair(g, carry):
        for b in range(2):
            ci = g * 2 + b

            @pl.when(ci + 1 < nchunks)
            def _():
                start_chunk(ci + 1, 1 - b)

            wait_chunk(b)
            process_chunk(b)
        return carry

    lax.fori_loop(0, nchunks // 2, chunk_pair, 0)

    @plsc.parallel_loop(0, NBINS // LANES, 1, unroll=4)
    def _merge(i):
        sl = pl.ds(i * LANES, LANES)
        h0[sl] = h0[sl] + h1[sl]
    pltpu.sync_copy(h0, out_hbm.at[wid])


def _combine_body(hist_ref, out_ref):
    h = hist_ref[...]
    c = jnp.sum(h, axis=0)
    cn = c[:HROWS, :]
    cp = c[HROWS:, :]
    m_tot = jnp.sum(cn)
    g_tot = jnp.sum(cp)
    n_tot = m_tot + g_tot

    row = lax.broadcasted_iota(jnp.int32, (HROWS, HCOLS), 0)
    col = lax.broadcasted_iota(jnp.int32, (HROWS, HCOLS), 1)
    b = (row * HCOLS + col).astype(jnp.float32)
    centers = -CLAMP + (b + 0.5) * (2.0 * CLAMP / NB)
    pcen = jax.nn.sigmoid(centers)

    s_pos = jnp.sum(cp * pcen)

    i1 = lax.broadcasted_iota(jnp.int32, (HCOLS, HCOLS), 0)
    j1 = lax.broadcasted_iota(jnp.int32, (HCOLS, HCOLS), 1)
    upper = (i1 <= j1).astype(jnp.float32)
    p_row = lax.dot_general(
        cn, upper, (((1,), (0,)), ((), ())),
        precision=lax.Precision.HIGHEST,
        preferred_element_type=jnp.float32,
    )
    rowsum = p_row[:, HCOLS - 1:HCOLS]
    i2 = lax.broadcasted_iota(jnp.int32, (HROWS, HROWS), 0)
    j2 = lax.broadcasted_iota(jnp.int32, (HROWS, HROWS), 1)
    strict_lower = (j2 < i2).astype(jnp.float32)
    offs = lax.dot_general(
        strict_lower, rowsum, (((1,), (0,)), ((), ())),
        precision=lax.Precision.HIGHEST,
        preferred_element_type=jnp.float32,
    )
    prefix = p_row + offs
    rank_above = m_tot - prefix

    denom = (g_tot + rank_above) * (g_tot + rank_above + cn)
    contrib = jnp.where(cn > 0.0, pcen * g_tot * cn / denom, 0.0)
    loss = 1.0 - s_pos / n_tot + jnp.sum(contrib)

    max_center = jnp.max(jnp.where(cn > 0.0, centers, -3.0 * CLAMP))
    loss = jnp.where(g_tot == 0.0, 1.0 + jax.nn.sigmoid(max_center), loss)
    out_ref[...] = jnp.reshape(loss, (1, 1))


@functools.cache
def _get_sc_hist():
    return pl.kernel(
        _sc_hist_body,
        out_type=jax.ShapeDtypeStruct((NW, NBINS), jnp.float32),
        mesh=plsc.VectorSubcoreMesh(core_axis_name="c", subcore_axis_name="s"),
        compiler_params=pltpu.CompilerParams(needs_layout_passes=False),
        scratch_types=[
            pltpu.VMEM((CH_ROWS, COLS), jnp.float32),
            pltpu.VMEM((CH_ROWS, COLS), jnp.float32),
            pltpu.VMEM((CH_ROWS, COLS), jnp.int32),
            pltpu.VMEM((CH_ROWS, COLS), jnp.int32),
            pltpu.VMEM((NBINS,), jnp.float32),
            pltpu.VMEM((NBINS,), jnp.float32),
            pltpu.SemaphoreType.DMA,
            pltpu.SemaphoreType.DMA,
            pltpu.SemaphoreType.DMA,
            pltpu.SemaphoreType.DMA,
        ],
    )


_combine = pl.pallas_call(
    _combine_body,
    out_shape=jax.ShapeDtypeStruct((1, 1), jnp.float32),
)


def kernel(inputs, targets):
    x = inputs.reshape(ROWS, COLS)
    t = targets.reshape(ROWS, COLS)
    hist = _get_sc_hist()(x, t)
    loss = _combine(hist.reshape(NW, 2 * HROWS, HCOLS))
    return loss.reshape(())


# speedup vs baseline: 1.1683x; 1.0148x over previous
import functools

import jax
import jax.numpy as jnp
from jax import lax
from jax.experimental import pallas as pl
from jax.experimental.pallas import tpu as pltpu
from jax.experimental.pallas import tpu_sc as plsc

NB = 12288
NBINS = 2 * NB
CLAMP = 12.0
SCALE = NB / (2.0 * CLAMP)
NW = 32
LANES = 16
UNROLL = 8

ROWS = 16384
COLS = 512
CH_ROWS = 32
CH_ELEMS = CH_ROWS * COLS
VPC = CH_ELEMS // LANES
VPR = COLS // LANES

HROWS = 96
HCOLS = 128


def _sc_hist_body(x_hbm, t_hbm, out_hbm, xb0, xb1, tb0, tb1, h0, h1,
                  sx0, sx1, st0, st1):
    wid = lax.axis_index("s") * 2 + lax.axis_index("c")
    rows_per_w = ROWS // NW
    row_base = wid * rows_per_w
    nchunks = rows_per_w // CH_ROWS
    xbufs = (xb0, xb1)
    tbufs = (tb0, tb1)
    sxs = (sx0, sx1)
    sts = (st0, st1)

    zeros = jnp.zeros((LANES,), jnp.float32)

    @plsc.parallel_loop(0, NBINS // LANES, 1, unroll=4)
    def _zero(i):
        h0[pl.ds(i * LANES, LANES)] = zeros
        h1[pl.ds(i * LANES, LANES)] = zeros

    ones = jnp.ones((LANES,), jnp.float32)
    hists = (h0, h1)

    def x_mul_add(xv):
        y = xv * SCALE + (CLAMP * SCALE)
        y = jnp.maximum(y, 0.0)
        return jnp.minimum(y, float(NB - 1))

    def start_chunk(cidx, b):
        row0 = row_base + cidx * CH_ROWS
        pltpu.make_async_copy(
            x_hbm.at[pl.ds(row0, CH_ROWS)], xbufs[b], sxs[b]).start()
        pltpu.make_async_copy(
            t_hbm.at[pl.ds(row0, CH_ROWS)], tbufs[b], sts[b]).start()

    def wait_chunk(b):
        pltpu.make_async_copy(
            x_hbm.at[pl.ds(row_base, CH_ROWS)], xbufs[b], sxs[b]).wait()
        pltpu.make_async_copy(
            t_hbm.at[pl.ds(row_base, CH_ROWS)], tbufs[b], sts[b]).wait()

    def process_chunk(b):
        xbuf = xbufs[b]
        tbuf = tbufs[b]

        @plsc.parallel_loop(0, VPC // UNROLL, 1, unroll=2)
        def _vecs(i):
            for u in range(UNROLL):
                v = i * UNROLL + u
                r = v // VPR
                c = (v % VPR) * LANES
                xv = xbuf[r, pl.ds(c, LANES)]
                tv = tbuf[r, pl.ds(c, LANES)]
                y = x_mul_add(xv)
                idx = y.astype(jnp.int32) + tv * NB
                plsc.addupdate_scatter(hists[u % 2], [idx], ones)

    start_chunk(0, 0)

    def chunk_pair(g, carry):
        for b in range(2):
            ci = g * 2 + b

            @pl.when(ci + 1 < nchunks)
            def _():
                start_chunk(ci + 1, 1 - b)

            wait_chunk(b)
            process_chunk(b)
        return carry

    lax.fori_loop(0, nchunks // 2, chunk_pair, 0)

    @plsc.parallel_loop(0, NBINS // LANES, 1, unroll=4)
    def _merge(i):
        sl = pl.ds(i * LANES, LANES)
        h0[sl] = h0[sl] + h1[sl]
    pltpu.sync_copy(h0, out_hbm.at[wid])


def _combine_body(hist_ref, out_ref):
    h = hist_ref[...]
    c = jnp.sum(h, axis=0)
    cn = c[:HROWS, :]
    cp = c[HROWS:, :]
    m_tot = jnp.sum(cn)
    g_tot = jnp.sum(cp)
    n_tot = m_tot + g_tot

    row = lax.broadcasted_iota(jnp.int32, (HROWS, HCOLS), 0)
    col = lax.broadcasted_iota(jnp.int32, (HROWS, HCOLS), 1)
    b = (row * HCOLS + col).astype(jnp.float32)
    centers = -CLAMP + (b + 0.5) * (2.0 * CLAMP / NB)
    pcen = jax.nn.sigmoid(centers)

    s_pos = jnp.sum(cp * pcen)

    i1 = lax.broadcasted_iota(jnp.int32, (HCOLS, HCOLS), 0)
    j1 = lax.broadcasted_iota(jnp.int32, (HCOLS, HCOLS), 1)
    upper = (i1 <= j1).astype(jnp.float32)
    p_row = lax.dot_general(
        cn, upper, (((1,), (0,)), ((), ())),
        precision=lax.Precision.HIGHEST,
        preferred_element_type=jnp.float32,
    )
    rowsum = p_row[:, HCOLS - 1:HCOLS]
    i2 = lax.broadcasted_iota(jnp.int32, (HROWS, HROWS), 0)
    j2 = lax.broadcasted_iota(jnp.int32, (HROWS, HROWS), 1)
    strict_lower = (j2 < i2).astype(jnp.float32)
    offs = lax.dot_general(
        strict_lower, rowsum, (((1,), (0,)), ((), ())),
        precision=lax.Precision.HIGHEST,
        preferred_element_type=jnp.float32,
    )
    prefix = p_row + offs
    rank_above = m_tot - prefix

    denom = (g_tot + rank_above) * (g_tot + rank_above + cn)
    contrib = jnp.where(cn > 0.0, pcen * g_tot * cn / denom, 0.0)
    loss = 1.0 - s_pos / n_tot + jnp.sum(contrib)

    max_center = jnp.max(jnp.where(cn > 0.0, centers, -3.0 * CLAMP))
    loss = jnp.where(g_tot == 0.0, 1.0 + jax.nn.sigmoid(max_center), loss)
    out_ref[...] = jnp.reshape(loss, (1, 1))


@functools.cache
def _get_sc_hist():
    return pl.kernel(
        _sc_hist_body,
        out_type=jax.ShapeDtypeStruct((NW, NBINS), jnp.float32),
        mesh=plsc.VectorSubcoreMesh(core_axis_name="c", subcore_axis_name="s"),
        compiler_params=pltpu.CompilerParams(needs_layout_passes=False),
        scratch_types=[
            pltpu.VMEM((CH_ROWS, COLS), jnp.float32),
            pltpu.VMEM((CH_ROWS, COLS), jnp.float32),
            pltpu.VMEM((CH_ROWS, COLS), jnp.int32),
            pltpu.VMEM((CH_ROWS, COLS), jnp.int32),
            pltpu.VMEM((NBINS,), jnp.float32),
            pltpu.VMEM((NBINS,), jnp.float32),
            pltpu.SemaphoreType.DMA,
            pltpu.SemaphoreType.DMA,
            pltpu.SemaphoreType.DMA,
            pltpu.SemaphoreType.DMA,
        ],
    )


_combine = pl.pallas_call(
    _combine_body,
    out_shape=jax.ShapeDtypeStruct((1, 1), jnp.float32),
)


def kernel(inputs, targets):
    x = inputs.reshape(ROWS, COLS)
    t = targets.reshape(ROWS, COLS)
    hist = _get_sc_hist()(x, t)
    loss = _combine(hist.reshape(NW, 2 * HROWS, HCOLS))
    return loss.reshape(())
